# baseline (speedup 1.0000x reference)
import functools

import jax
import jax.numpy as jnp
from jax import lax
from jax.experimental import pallas as pl
from jax.experimental.pallas import tpu as pltpu
from jax.experimental.pallas import tpu_sc as plsc


_CHUNK = 80


@functools.cache
def _gather_kernel(V, D, Bf):
    info = plsc.get_sparse_core_info()
    nw = info.num_cores * info.num_subcores
    b_per_w = Bf // nw
    assert Bf % nw == 0 and b_per_w % _CHUNK == 0
    n_chunks = b_per_w // _CHUNK
    mesh = plsc.VectorSubcoreMesh(core_axis_name="c", subcore_axis_name="s")

    @functools.partial(
        pl.kernel,
        mesh=mesh,
        out_type=jax.ShapeDtypeStruct((Bf, D), jnp.float32),
        scratch_types=[
            pltpu.VMEM((b_per_w,), jnp.int32),
            pltpu.VMEM((b_per_w, D), jnp.float32),
            pltpu.SemaphoreType.DMA,
        ],
        compiler_params=pltpu.CompilerParams(use_tc_tiling_on_sc=False),
    )
    def gather_k(table_hbm, idx_hbm, out_hbm, idx_v, rows_v, sem):
        wid = lax.axis_index("s") * info.num_cores + lax.axis_index("c")
        base = wid * b_per_w
        pltpu.sync_copy(idx_hbm.at[pl.ds(base, b_per_w)], idx_v)
        copies = []
        for j in range(n_chunks):
            copies.append(
                pltpu.async_copy(
                    table_hbm.at[idx_v.at[pl.ds(j * _CHUNK, _CHUNK)]],
                    rows_v.at[pl.ds(j * _CHUNK, _CHUNK), :],
                    sem,
                )
            )
        for c in copies:
            c.wait()
        pltpu.sync_copy(rows_v, out_hbm.at[pl.ds(base, b_per_w)])

    return gather_k


def _embed_gather(emb, idx):
    V, D = emb.shape
    return _gather_kernel(V, D, idx.shape[0])(emb, idx)


def _transpose_body(et_ref, o_ref):
    d = et_ref.shape[0]
    half = et_ref.shape[1] // 2
    o_ref[:, :d] = et_ref[:, :half].T
    o_ref[:, d:] = et_ref[:, half:].T


_TE = 12800


def _emb_rowmajor(embT):
    D, V = embT.shape
    vp = _TE * pl.cdiv(V, _TE)
    return pl.pallas_call(
        _transpose_body,
        grid=(vp // _TE,),
        in_specs=[pl.BlockSpec((D, _TE), lambda j: (0, j))],
        out_specs=pl.BlockSpec((_TE // 2, 2 * D), lambda j: (j, 0)),
        out_shape=jax.ShapeDtypeStruct((vp // 2, 2 * D), jnp.float32),
        compiler_params=pltpu.CompilerParams(
            dimension_semantics=("arbitrary",)
        ),
    )(embT)


def _mlp12_body(x_ref, w1_ref, b1_ref, w2_ref, b2_ref, o_ref):
    h = jnp.dot(x_ref[...], w1_ref[...], preferred_element_type=jnp.float32)
    h = jnp.maximum(h + b1_ref[...], 0.0)
    h2 = jnp.dot(h, w2_ref[...], preferred_element_type=jnp.float32)
    o_ref[...] = jnp.maximum(h2 + b2_ref[...], 0.0).T


def _mlp12T(x, W1, b1, W2, b2):
    B, CD = x.shape
    H = W1.shape[1]
    BB = 512
    return pl.pallas_call(
        _mlp12_body,
        grid=(B // BB,),
        in_specs=[
            pl.BlockSpec((BB, CD), lambda i: (i, 0)),
            pl.BlockSpec((CD, H), lambda i: (0, 0)),
            pl.BlockSpec((1, H), lambda i: (0, 0)),
            pl.BlockSpec((H, H), lambda i: (0, 0)),
            pl.BlockSpec((1, H), lambda i: (0, 0)),
        ],
        out_specs=pl.BlockSpec((H, BB), lambda i: (0, i)),
        out_shape=jax.ShapeDtypeStruct((H, B), jnp.float32),
        compiler_params=pltpu.CompilerParams(
            dimension_semantics=("arbitrary",)
        ),
    )(x, W1, b1.reshape(1, -1), W2, b2.reshape(1, -1))


_TV = 4096


def _projT_body(ht_ref, w3t_ref, b3_ref, o_ref):
    acc = jnp.dot(w3t_ref[...], ht_ref[...], preferred_element_type=jnp.float32)
    o_ref[...] = acc + b3_ref[...].T


def _projT(ht, W3T, b3):
    H, B = ht.shape
    V = W3T.shape[0]
    return pl.pallas_call(
        _projT_body,
        grid=(pl.cdiv(V, _TV),),
        in_specs=[
            pl.BlockSpec((H, B), lambda j: (0, 0)),
            pl.BlockSpec((_TV, H), lambda j: (j, 0)),
            pl.BlockSpec((1, _TV), lambda j: (0, j)),
        ],
        out_specs=pl.BlockSpec((_TV, B), lambda j: (j, 0)),
        out_shape=jax.ShapeDtypeStruct((V, B), jnp.float32),
        compiler_params=pltpu.CompilerParams(
            dimension_semantics=("arbitrary",),
            vmem_limit_bytes=60000 * 1024,
        ),
    )(ht, W3T, b3.reshape(1, -1))


def kernel(inputs, emb, W1, b1, W2, b2, W3, b3):
    B, C = inputs.shape
    idx = inputs.astype(jnp.int32).reshape(-1)
    emb2 = _emb_rowmajor(emb.T)
    emb_rm = emb2.reshape(emb2.shape[0] * 2, emb.shape[1])
    half = _TE // 2
    g = (idx // _TE) * _TE + (idx % half) * 2 + (idx % _TE) // half
    rows = _embed_gather(emb_rm, g)
    x = rows.reshape(B, C * emb.shape[1])
    h2t = _mlp12T(x, W1, b1, W2, b2)
    outT = _projT(h2t, W3.T, b3)
    return outT.T


# speedup vs baseline: 1.0045x; 1.0045x over previous
import functools

import jax
import jax.numpy as jnp
from jax import lax
from jax.experimental import pallas as pl
from jax.experimental.pallas import tpu as pltpu
from jax.experimental.pallas import tpu_sc as plsc


_CHUNK = 80


@functools.cache
def _gather_kernel(V, D, Bf):
    info = plsc.get_sparse_core_info()
    nw = info.num_cores * info.num_subcores
    b_per_w = Bf // nw
    assert Bf % nw == 0 and b_per_w % _CHUNK == 0
    n_chunks = b_per_w // _CHUNK
    mesh = plsc.VectorSubcoreMesh(core_axis_name="c", subcore_axis_name="s")

    @functools.partial(
        pl.kernel,
        mesh=mesh,
        out_type=jax.ShapeDtypeStruct((Bf, D), jnp.float32),
        scratch_types=[
            pltpu.VMEM((b_per_w,), jnp.int32),
            pltpu.VMEM((b_per_w, D), jnp.float32),
            pltpu.SemaphoreType.DMA,
        ],
        compiler_params=pltpu.CompilerParams(use_tc_tiling_on_sc=False),
    )
    def gather_k(table_hbm, idx_hbm, out_hbm, idx_v, rows_v, sem):
        wid = lax.axis_index("s") * info.num_cores + lax.axis_index("c")
        base = wid * b_per_w
        pltpu.sync_copy(idx_hbm.at[pl.ds(base, b_per_w)], idx_v)
        copies = []
        for j in range(n_chunks):
            copies.append(
                pltpu.async_copy(
                    table_hbm.at[idx_v.at[pl.ds(j * _CHUNK, _CHUNK)]],
                    rows_v.at[pl.ds(j * _CHUNK, _CHUNK), :],
                    sem,
                )
            )
        for c in copies:
            c.wait()
        pltpu.sync_copy(rows_v, out_hbm.at[pl.ds(base, b_per_w)])

    return gather_k


def _embed_gather(emb, idx):
    V, D = emb.shape
    return _gather_kernel(V, D, idx.shape[0])(emb, idx)


def _transpose_body(et_ref, o_ref):
    d = et_ref.shape[0]
    half = et_ref.shape[1] // 2
    o_ref[:, :d] = et_ref[:, :half].T
    o_ref[:, d:] = et_ref[:, half:].T


_TE = 16384


def _emb_rowmajor(embT):
    D, V = embT.shape
    vp = _TE * pl.cdiv(V, _TE)
    return pl.pallas_call(
        _transpose_body,
        grid=(vp // _TE,),
        in_specs=[pl.BlockSpec((D, _TE), lambda j: (0, j))],
        out_specs=pl.BlockSpec((_TE // 2, 2 * D), lambda j: (j, 0)),
        out_shape=jax.ShapeDtypeStruct((vp // 2, 2 * D), jnp.float32),
        compiler_params=pltpu.CompilerParams(
            dimension_semantics=("arbitrary",)
        ),
    )(embT)


def _mlp12_body(x_ref, w1_ref, b1_ref, w2_ref, b2_ref, o_ref):
    h = jnp.dot(x_ref[...], w1_ref[...], preferred_element_type=jnp.float32)
    h = jnp.maximum(h + b1_ref[...], 0.0)
    h2 = jnp.dot(h, w2_ref[...], preferred_element_type=jnp.float32)
    o_ref[...] = jnp.maximum(h2 + b2_ref[...], 0.0).T


def _mlp12T(x, W1, b1, W2, b2):
    B, CD = x.shape
    H = W1.shape[1]
    BB = 512
    return pl.pallas_call(
        _mlp12_body,
        grid=(B // BB,),
        in_specs=[
            pl.BlockSpec((BB, CD), lambda i: (i, 0)),
            pl.BlockSpec((CD, H), lambda i: (0, 0)),
            pl.BlockSpec((1, H), lambda i: (0, 0)),
            pl.BlockSpec((H, H), lambda i: (0, 0)),
            pl.BlockSpec((1, H), lambda i: (0, 0)),
        ],
        out_specs=pl.BlockSpec((H, BB), lambda i: (0, i)),
        out_shape=jax.ShapeDtypeStruct((H, B), jnp.float32),
        compiler_params=pltpu.CompilerParams(
            dimension_semantics=("arbitrary",)
        ),
    )(x, W1, b1.reshape(1, -1), W2, b2.reshape(1, -1))


_TV = 4096


def _projT_body(ht_ref, w3t_ref, b3_ref, o_ref):
    acc = jnp.dot(w3t_ref[...], ht_ref[...], preferred_element_type=jnp.float32)
    o_ref[...] = acc + b3_ref[...].T


def _projT(ht, W3T, b3):
    H, B = ht.shape
    V = W3T.shape[0]
    return pl.pallas_call(
        _projT_body,
        grid=(pl.cdiv(V, _TV),),
        in_specs=[
            pl.BlockSpec((H, B), lambda j: (0, 0)),
            pl.BlockSpec((_TV, H), lambda j: (j, 0)),
            pl.BlockSpec((1, _TV), lambda j: (0, j)),
        ],
        out_specs=pl.BlockSpec((_TV, B), lambda j: (j, 0)),
        out_shape=jax.ShapeDtypeStruct((V, B), jnp.float32),
        compiler_params=pltpu.CompilerParams(
            dimension_semantics=("arbitrary",),
            vmem_limit_bytes=60000 * 1024,
        ),
    )(ht, W3T, b3.reshape(1, -1))


def kernel(inputs, emb, W1, b1, W2, b2, W3, b3):
    B, C = inputs.shape
    idx = inputs.astype(jnp.int32).reshape(-1)
    emb2 = _emb_rowmajor(emb.T)
    emb_rm = emb2.reshape(emb2.shape[0] * 2, emb.shape[1])
    half = _TE // 2
    g = (idx // _TE) * _TE + (idx % half) * 2 + (idx % _TE) // half
    rows = _embed_gather(emb_rm, g)
    x = rows.reshape(B, C * emb.shape[1])
    h2t = _mlp12T(x, W1, b1, W2, b2)
    outT = _projT(h2t, W3.T, b3)
    return outT.T
